# baseline (speedup 1.0000x reference)
import functools

import jax
import jax.numpy as jnp
from jax import lax
from jax.experimental import pallas as pl
from jax.experimental.pallas import tpu as pltpu
from jax.experimental.pallas import tpu_sc as plsc

N = 8192
M = 8192
BIG = 3.0e38
FAR = 1.0e18

NW = 32
BPW = N // NW
L = 16
RV = M // L
UNROLL = 8


def _sc_main(bx_hbm, by_hbm, rx_hbm, ry_hbm,
             rowmin_hbm, maskf_hbm, colpart_hbm,
             rx_v, ry_v, colacc_v, bx_v, by_v, rmin_v, mask_v):
    wid = lax.axis_index("c") * 16 + lax.axis_index("s")
    base = wid * BPW

    pltpu.sync_copy(rx_hbm, rx_v)
    pltpu.sync_copy(ry_hbm, ry_v)
    pltpu.sync_copy(bx_hbm.at[pl.ds(base, BPW)], bx_v)
    pltpu.sync_copy(by_hbm.at[pl.ds(base, BPW)], by_v)

    def mask_body(t, carry):
        sl = pl.ds(t * L, L)
        bxv = bx_v[sl]
        byv = by_v[sl]
        ok = ((bxv >= -2000.0) & (bxv <= 2000.0) &
              (byv >= -2000.0) & (byv <= 2000.0))
        mask_v[sl] = jnp.where(ok, jnp.float32(1.0), jnp.float32(0.0))
        bx_v[sl] = jnp.where(ok, bxv, jnp.float32(FAR))
        by_v[sl] = jnp.where(ok, byv, jnp.float32(FAR))
        return carry

    lax.fori_loop(0, BPW // L, mask_body, 0)

    def init_body(j, carry):
        colacc_v[pl.ds(j * L, L)] = jnp.full((L,), BIG, jnp.float32)
        return carry

    lax.fori_loop(0, RV, init_body, 0)

    def super_body(t, carry):
        i0 = t * L
        bxv16 = bx_v[pl.ds(i0, L)]
        byv16 = by_v[pl.ds(i0, L)]
        for sub in range(4):
            bxs = []
            bys = []
            for k in range(4):
                lane = sub * 4 + k
                bxs.append(jnp.full((L,), bxv16[lane], jnp.float32))
                bys.append(jnp.full((L,), byv16[lane], jnp.float32))

            def inner(jj, accs):
                accs = list(accs)
                for u in range(UNROLL):
                    sl = pl.ds((jj * UNROLL + u) * L, L)
                    rxv = rx_v[sl]
                    ryv = ry_v[sl]
                    d2s = []
                    for k in range(4):
                        dx = rxv - bxs[k]
                        dy = ryv - bys[k]
                        d2 = dx * dx + dy * dy
                        accs[k] = jnp.minimum(accs[k], d2)
                        d2s.append(d2)
                    cm = jnp.minimum(jnp.minimum(d2s[0], d2s[1]),
                                     jnp.minimum(d2s[2], d2s[3]))
                    colacc_v[sl] = jnp.minimum(colacc_v[sl], cm)
                return tuple(accs)

            accs = lax.fori_loop(
                0, RV // UNROLL, inner,
                tuple(jnp.full((L,), BIG, jnp.float32) for _ in range(4)))
            for k in range(4):
                rmin_v[pl.ds((i0 + sub * 4 + k) * L, L)] = accs[k]
        return carry

    lax.fori_loop(0, BPW // L, super_body, 0)

    pltpu.sync_copy(rmin_v, rowmin_hbm.at[pl.ds(base * L, BPW * L)])
    pltpu.sync_copy(mask_v, maskf_hbm.at[pl.ds(base, BPW)])
    pltpu.sync_copy(colacc_v, colpart_hbm.at[wid])


def _make_sc_call():
    return functools.partial(
        pl.kernel,
        out_type=(jax.ShapeDtypeStruct((N * L,), jnp.float32),
                  jax.ShapeDtypeStruct((N,), jnp.float32),
                  jax.ShapeDtypeStruct((NW, M), jnp.float32)),
        mesh=plsc.VectorSubcoreMesh(core_axis_name="c", subcore_axis_name="s",
                                    num_cores=2, num_subcores=16),
        scratch_types=[
            pltpu.VMEM((M,), jnp.float32),
            pltpu.VMEM((M,), jnp.float32),
            pltpu.VMEM((M,), jnp.float32),
            pltpu.VMEM((BPW,), jnp.float32),
            pltpu.VMEM((BPW,), jnp.float32),
            pltpu.VMEM((BPW * L,), jnp.float32),
            pltpu.VMEM((BPW,), jnp.float32),
        ],
    )(_sc_main)


def _finalize_body(rm2p_ref, mk_ref, colpart_ref, out_ref):
    colmin2 = jnp.min(colpart_ref[...], axis=0, keepdims=True)
    sum2 = jnp.sum(jnp.sqrt(colmin2))
    mk = mk_ref[...]
    rm2 = jnp.min(rm2p_ref[...], axis=1, keepdims=True)
    rd = jnp.sqrt(rm2) * mk
    sum1 = jnp.sum(rd)
    cnt = jnp.sum(mk)
    mean1 = sum1 / jnp.maximum(cnt, 1.0)
    mean2 = sum2 / jnp.float32(M)
    out_ref[0, 0] = (mean1 + mean2) * 0.5


@jax.jit
def _centerline_loss(bez, ref):
    bx = bez[:, 0]
    by = bez[:, 1]
    rx = ref[:, 1]
    ry = ref[:, 0]

    rowmin2, maskf, colpart = _make_sc_call()(bx, by, rx, ry)

    out = pl.pallas_call(
        _finalize_body,
        out_specs=pl.BlockSpec(memory_space=pltpu.SMEM),
        out_shape=jax.ShapeDtypeStruct((1, 1), jnp.float32),
    )(rowmin2.reshape(N, L), maskf.reshape(N, 1), colpart)
    return out[0, 0]


def kernel(bezier_proj_centerline_img, ref_catheter_centerline):
    return _centerline_loss(bezier_proj_centerline_img,
                            ref_catheter_centerline)
